# Initial kernel scaffold; baseline (speedup 1.0000x reference)
import jax
import jax.numpy as jnp
from jax.experimental import pallas as pl


def kernel(x, table):
    raise NotImplementedError("write your pallas kernel here")


# speedup vs baseline: 1.0794x; 1.0794x over previous
import functools

import jax
import jax.numpy as jnp
from jax import lax
from jax.experimental import pallas as pl
from jax.experimental.pallas import tpu as pltpu
from jax.experimental.pallas import tpu_sc as plsc

_NUM_BUCKETS = 1000000
_D = 32
_LANES = 16
_SUB = 128


def _make_gather(n: int, chunk: int):
    info = plsc.get_sparse_core_info()
    nc, ns = info.num_cores, info.num_subcores
    nw = nc * ns
    per_w = n // nw
    n_chunks = per_w // chunk
    assert per_w * nw == n and n_chunks * chunk == per_w

    mesh = plsc.VectorSubcoreMesh(core_axis_name="c", subcore_axis_name="s")

    @functools.partial(
        pl.kernel,
        mesh=mesh,
        compiler_params=pltpu.CompilerParams(use_tc_tiling_on_sc=False),
        out_type=jax.ShapeDtypeStruct((n, _D), jnp.float32),
        scratch_types=[
            pltpu.VMEM((chunk,), jnp.int32),
            pltpu.VMEM((chunk, _D), jnp.float32),
            pltpu.SemaphoreType.DMA,
        ],
    )
    def k(x_hbm, table_hbm, out_hbm, idx_v, rows_v, sem):
        wid = lax.axis_index("s") * nc + lax.axis_index("c")
        base = wid * per_w

        def chunk_body(c, carry):
            start = base + c * chunk
            pltpu.sync_copy(x_hbm.at[pl.ds(start, chunk)], idx_v)

            def mod_body(i, carry2):
                v = idx_v[pl.ds(i * _LANES, _LANES)]
                idx_v[pl.ds(i * _LANES, _LANES)] = lax.rem(v, _NUM_BUCKETS)
                return carry2

            lax.fori_loop(0, chunk // _LANES, mod_body, 0, unroll=4)

            copies = [
                pltpu.async_copy(
                    table_hbm.at[idx_v.at[pl.ds(j * _SUB, _SUB)]],
                    rows_v.at[pl.ds(j * _SUB, _SUB)],
                    sem,
                )
                for j in range(chunk // _SUB)
            ]
            for cp in copies:
                cp.wait()
            pltpu.sync_copy(rows_v, out_hbm.at[pl.ds(start, chunk)])
            return carry

        lax.fori_loop(0, n_chunks, chunk_body, 0)

    return k


def kernel(x, table):
    b, f = x.shape
    n = b * f
    xf = x.reshape(n).astype(jnp.int32)
    out = _make_gather(n, 1024)(xf, table)
    return out.reshape(b, f, _D)


# speedup vs baseline: 1.0976x; 1.0169x over previous
import functools

import jax
import jax.numpy as jnp
from jax import lax
from jax.experimental import pallas as pl
from jax.experimental.pallas import tpu as pltpu
from jax.experimental.pallas import tpu_sc as plsc

_NUM_BUCKETS = 1000000
_D = 32
_LANES = 16
_SUB = 128
_NBUF = 4


def _make_gather(n: int, chunk: int):
    info = plsc.get_sparse_core_info()
    nc, ns = info.num_cores, info.num_subcores
    nw = nc * ns
    per_w = n // nw
    n_chunks = per_w // chunk
    assert per_w * nw == n and n_chunks * chunk == per_w
    assert n_chunks % _NBUF == 0 and n_chunks > _NBUF
    assert chunk % _SUB == 0

    mesh = plsc.VectorSubcoreMesh(core_axis_name="c", subcore_axis_name="s")

    @functools.partial(
        pl.kernel,
        mesh=mesh,
        compiler_params=pltpu.CompilerParams(use_tc_tiling_on_sc=False),
        out_type=jax.ShapeDtypeStruct((n, _D), jnp.float32),
        scratch_types=(
            [pltpu.VMEM((chunk,), jnp.int32) for _ in range(_NBUF)]
            + [pltpu.VMEM((chunk, _D), jnp.float32) for _ in range(_NBUF)]
            + [pltpu.SemaphoreType.DMA for _ in range(2 * _NBUF)]
        ),
    )
    def k(x_hbm, table_hbm, out_hbm, *bufs):
        idx_v = bufs[:_NBUF]
        rows_v = bufs[_NBUF:2 * _NBUF]
        gsem = bufs[2 * _NBUF:3 * _NBUF]
        wsem = bufs[3 * _NBUF:4 * _NBUF]

        wid = lax.axis_index("s") * nc + lax.axis_index("c")
        base = wid * per_w

        def load_and_hash(g, b):
            pltpu.sync_copy(x_hbm.at[pl.ds(base + g * chunk, chunk)], idx_v[b])

            def mod_body(i, carry):
                v = idx_v[b][pl.ds(i * _LANES, _LANES)]
                idx_v[b][pl.ds(i * _LANES, _LANES)] = lax.rem(v, _NUM_BUCKETS)
                return carry

            lax.fori_loop(0, chunk // _LANES, mod_body, 0, unroll=4)

        def start_gather(b):
            for j in range(chunk // _SUB):
                pltpu.async_copy(
                    table_hbm.at[idx_v[b].at[pl.ds(j * _SUB, _SUB)]],
                    rows_v[b].at[pl.ds(j * _SUB, _SUB)],
                    gsem[b],
                )

        def wait_gather(b):
            for j in range(chunk // _SUB):
                pltpu.make_async_copy(
                    table_hbm.at[idx_v[b].at[pl.ds(j * _SUB, _SUB)]],
                    rows_v[b].at[pl.ds(j * _SUB, _SUB)],
                    gsem[b],
                ).wait()

        def wait_writeback(b):
            pltpu.make_async_copy(
                rows_v[b], out_hbm.at[pl.ds(base, chunk)], wsem[b]
            ).wait()

        for b in range(_NBUF):
            load_and_hash(b, b)
            start_gather(b)

        def outer(oo, carry):
            o = oo * _NBUF
            for b in range(_NBUF):
                g = o + b
                wait_gather(b)
                pltpu.async_copy(
                    rows_v[b],
                    out_hbm.at[pl.ds(base + g * chunk, chunk)],
                    wsem[b],
                )

                @pl.when(o < n_chunks - _NBUF)
                def _():
                    load_and_hash(g + _NBUF, b)
                    wait_writeback(b)
                    start_gather(b)

            return carry

        lax.fori_loop(0, n_chunks // _NBUF, outer, 0, unroll=False)

        for b in range(_NBUF):
            wait_writeback(b)

    return k


def kernel(x, table):
    b, f = x.shape
    n = b * f
    xf = x.reshape(n).astype(jnp.int32)
    out = _make_gather(n, 640)(xf, table)
    return out.reshape(b, f, _D)


# speedup vs baseline: 3.6567x; 3.3315x over previous
import functools

import jax
import jax.numpy as jnp
from jax import lax
from jax.experimental import pallas as pl
from jax.experimental.pallas import tpu as pltpu
from jax.experimental.pallas import tpu_sc as plsc

_NUM_BUCKETS = 1000000
_D = 32
_LANES = 16
_NBUF = 4


def _make_gather(n: int, chunk: int):
    info = plsc.get_sparse_core_info()
    nc, ns = info.num_cores, info.num_subcores
    nw = nc * ns
    per_w = n // nw
    n_chunks = per_w // chunk
    assert per_w * nw == n and n_chunks * chunk == per_w
    assert n_chunks % _NBUF == 0 and n_chunks > _NBUF

    mesh = plsc.VectorSubcoreMesh(core_axis_name="c", subcore_axis_name="s")

    @functools.partial(
        pl.kernel,
        mesh=mesh,
        compiler_params=pltpu.CompilerParams(use_tc_tiling_on_sc=False),
        out_type=jax.ShapeDtypeStruct((n * _D,), jnp.float32),
        scratch_types=(
            [pltpu.VMEM((chunk,), jnp.int32) for _ in range(_NBUF)]
            + [pltpu.VMEM((chunk, _D), jnp.float32) for _ in range(_NBUF)]
            + [pltpu.VMEM((chunk * _D,), jnp.float32) for _ in range(_NBUF)]
            + [pltpu.SemaphoreType.DMA for _ in range(2 * _NBUF)]
        ),
    )
    def k(x_hbm, table_hbm, out_hbm, *bufs):
        idx_v = bufs[:_NBUF]
        rows_v = bufs[_NBUF:2 * _NBUF]
        flat_v = bufs[2 * _NBUF:3 * _NBUF]
        gsem = bufs[3 * _NBUF:4 * _NBUF]
        wsem = bufs[4 * _NBUF:5 * _NBUF]

        wid = lax.axis_index("s") * nc + lax.axis_index("c")
        base = wid * per_w

        def load_and_hash(g, b):
            pltpu.sync_copy(x_hbm.at[pl.ds(base + g * chunk, chunk)], idx_v[b])

            def mod_body(i, carry):
                v = idx_v[b][pl.ds(i * _LANES, _LANES)]
                idx_v[b][pl.ds(i * _LANES, _LANES)] = lax.rem(v, _NUM_BUCKETS)
                return carry

            lax.fori_loop(0, chunk // _LANES, mod_body, 0, unroll=4)

        def start_gather(b):
            pltpu.async_copy(table_hbm.at[idx_v[b]], rows_v[b], gsem[b])

        def wait_gather(b):
            pltpu.make_async_copy(
                table_hbm.at[idx_v[b]], rows_v[b], gsem[b]).wait()

        def repack(b):
            def body(i, carry):
                lo = rows_v[b][i, pl.ds(0, _LANES)]
                hi = rows_v[b][i, pl.ds(_LANES, _LANES)]
                flat_v[b][pl.ds(i * _D, _LANES)] = lo
                flat_v[b][pl.ds(i * _D + _LANES, _LANES)] = hi
                return carry

            lax.fori_loop(0, chunk, body, 0, unroll=8)

        def wait_writeback(b):
            pltpu.make_async_copy(
                flat_v[b], out_hbm.at[pl.ds(base * _D, chunk * _D)],
                wsem[b]).wait()

        for b in range(_NBUF):
            load_and_hash(b, b)
            start_gather(b)

        def outer(oo, carry):
            o = oo * _NBUF
            for b in range(_NBUF):
                g = o + b
                wait_gather(b)
                repack(b)
                pltpu.async_copy(
                    flat_v[b],
                    out_hbm.at[pl.ds((base + g * chunk) * _D, chunk * _D)],
                    wsem[b])

                @pl.when(o < n_chunks - _NBUF)
                def _():
                    load_and_hash(g + _NBUF, b)
                    wait_writeback(b)
                    start_gather(b)

            return carry

        lax.fori_loop(0, n_chunks // _NBUF, outer, 0, unroll=False)

        for b in range(_NBUF):
            wait_writeback(b)

    return k


def kernel(x, table):
    b, f = x.shape
    n = b * f
    xf = x.reshape(n).astype(jnp.int32)
    out = _make_gather(n, 320)(xf, table)
    return out.reshape(b, f, _D)
